# Initial kernel scaffold; baseline (speedup 1.0000x reference)
import jax
import jax.numpy as jnp
from jax.experimental import pallas as pl


def kernel(x_nodes, node_type, edge_index, rbf_edges, dist, emb_table, W_edge, b_edge, W_comb, b_comb):
    raise NotImplementedError("write your pallas kernel here")


# speedup vs baseline: 1.8582x; 1.8582x over previous
import functools

import jax
import jax.numpy as jnp
import numpy as np
from jax import lax
from jax.experimental import pallas as pl
from jax.experimental.pallas import tpu as pltpu
from jax.experimental.pallas import tpu_sc as plsc

_CUTOFF = 5.0
_NC = 2
_NS = 16
_NW = _NC * _NS
_CH = 128


def _tgather_kernel(EW, N, nt_hbm, src_hbm, t_hbm, nt_v, src_v, t_v):
    c = lax.axis_index("c")
    s = lax.axis_index("s")
    w = s * _NC + c
    base = w * EW
    pltpu.sync_copy(nt_hbm, nt_v)
    pltpu.sync_copy(src_hbm.at[pl.ds(base, EW)], src_v)

    def body(i, carry):
        idx = src_v[pl.ds(i * 16, 16)]
        t_v[pl.ds(i * 16, 16)] = plsc.load_gather(nt_v, [idx])
        return carry

    lax.fori_loop(0, EW // 16, body, 0)
    pltpu.sync_copy(t_v, t_hbm.at[pl.ds(base, EW)])


def _msg_kernel(BE, rbf_ref, t_ref, dist_ref, WeT_ref, be_ref, embp_ref, out_ref):
    t = t_ref[...]
    onehot = (t == lax.broadcasted_iota(jnp.int32, (BE, 128), 1)).astype(jnp.float32)
    emb_rows = jnp.dot(onehot, embp_ref[...], preferred_element_type=jnp.float32)
    xe = jnp.dot(rbf_ref[...], WeT_ref[...], preferred_element_type=jnp.float32)
    xe = xe + be_ref[...]
    d = dist_ref[...]
    w = 0.5 * (jnp.cos(d * (np.pi / _CUTOFF)) + 1.0)
    w = w * (d < _CUTOFF).astype(jnp.float32)
    out_ref[...] = emb_rows * (xe * w)


def _scatter_kernel(E_pad, N, D, msg_hbm, dst_hbm, z_hbm, out_hbm, dst_v, msg_v, acc):
    c = lax.axis_index("c")
    s = lax.axis_index("s")
    zr = 624
    tail = N - zr * _NS

    pltpu.sync_copy(z_hbm.at[pl.ds(s * zr, zr)], acc.at[pl.ds(s * zr, zr)])

    @pl.when(s == 0)
    def _():
        pltpu.sync_copy(z_hbm.at[pl.ds(_NS * zr, tail)],
                        acc.at[pl.ds(_NS * zr, tail)])

    plsc.subcore_barrier()

    half = E_pad // _NC
    per_tile = half // _NS
    n_chunks = per_tile // _CH
    tile_base = c * half + s * per_tile

    def body(k, carry):
        base = tile_base + k * _CH
        pltpu.sync_copy(dst_hbm.at[pl.ds(base, _CH)], dst_v)
        pltpu.sync_copy(msg_hbm.at[pl.ds(base, _CH)], msg_v)
        pltpu.sync_copy(msg_v, acc.at[dst_v], add=True)
        return carry

    lax.fori_loop(0, n_chunks, body, 0)
    plsc.subcore_barrier()

    pltpu.sync_copy(acc.at[pl.ds(s * zr, zr)],
                    out_hbm.at[pl.ds(c * N + s * zr, zr)])

    @pl.when(s == 0)
    def _():
        pltpu.sync_copy(acc.at[pl.ds(_NS * zr, tail)],
                        out_hbm.at[pl.ds(c * N + _NS * zr, tail)])


def _comb_kernel(xn_ref, p0_ref, p1_ref, Wc1T_ref, Wc2T_ref, bc_ref, out_ref):
    ps = p0_ref[...] + p1_ref[...]
    out_ref[...] = (
        jnp.dot(xn_ref[...], Wc1T_ref[...], preferred_element_type=jnp.float32)
        + jnp.dot(ps, Wc2T_ref[...], preferred_element_type=jnp.float32)
        + bc_ref[...]
    )


def kernel(x_nodes, node_type, edge_index, rbf_edges, dist, emb_table,
           W_edge, b_edge, W_comb, b_comb):
    N, D = x_nodes.shape
    E = edge_index.shape[1]
    n_rbf = rbf_edges.shape[1]
    n_elem = emb_table.shape[0]

    gran = _NW * _CH
    E_pad = ((E + gran - 1) // gran) * gran
    pad = E_pad - E
    EW = E_pad // _NW

    src = jnp.concatenate([edge_index[0], jnp.zeros((pad,), edge_index.dtype)])
    dst = jnp.concatenate([edge_index[1], jnp.zeros((pad,), edge_index.dtype)])
    rbf_p = jnp.concatenate(
        [rbf_edges, jnp.zeros((pad, n_rbf), rbf_edges.dtype)])
    dist_p = jnp.concatenate(
        [dist, jnp.full((pad,), 2.0 * _CUTOFF, dist.dtype)])
    emb_pad = jnp.zeros((128, D), emb_table.dtype).at[:n_elem].set(emb_table)

    mesh = plsc.VectorSubcoreMesh(core_axis_name="c", subcore_axis_name="s")

    t = pl.kernel(
        functools.partial(_tgather_kernel, EW, N),
        out_type=jax.ShapeDtypeStruct((E_pad,), jnp.int32),
        mesh=mesh,
        scratch_types=[
            pltpu.VMEM((N,), jnp.int32),
            pltpu.VMEM((EW,), jnp.int32),
            pltpu.VMEM((EW,), jnp.int32),
        ],
        compiler_params=pltpu.CompilerParams(needs_layout_passes=False),
    )(node_type.astype(jnp.int32), src.astype(jnp.int32))

    BE = 2048
    n_be = E_pad // BE
    msg = pl.pallas_call(
        functools.partial(_msg_kernel, BE),
        grid=(n_be,),
        in_specs=[
            pl.BlockSpec((BE, n_rbf), lambda i: (i, 0)),
            pl.BlockSpec((BE, 1), lambda i: (i, 0)),
            pl.BlockSpec((BE, 1), lambda i: (i, 0)),
            pl.BlockSpec((n_rbf, D), lambda i: (0, 0)),
            pl.BlockSpec((1, D), lambda i: (0, 0)),
            pl.BlockSpec((128, D), lambda i: (0, 0)),
        ],
        out_specs=pl.BlockSpec((BE, D), lambda i: (i, 0)),
        out_shape=jax.ShapeDtypeStruct((E_pad, D), jnp.float32),
    )(rbf_p, t.reshape(E_pad, 1), dist_p.reshape(E_pad, 1),
      W_edge.T, b_edge.reshape(1, D), emb_pad)

    partials = pl.kernel(
        functools.partial(_scatter_kernel, E_pad, N, D),
        out_type=jax.ShapeDtypeStruct((_NC * N, D), jnp.float32),
        mesh=mesh,
        scratch_types=[
            pltpu.VMEM((_CH,), jnp.int32),
            pltpu.VMEM((_CH, D), jnp.float32),
            pltpu.VMEM_SHARED((N, D), jnp.float32),
        ],
    )(msg, dst.astype(jnp.int32), jnp.zeros((N, D), jnp.float32))

    BN = 2000
    out = pl.pallas_call(
        _comb_kernel,
        grid=(N // BN,),
        in_specs=[
            pl.BlockSpec((BN, D), lambda i: (i, 0)),
            pl.BlockSpec((BN, D), lambda i: (i, 0)),
            pl.BlockSpec((BN, D), lambda i: (i, 0)),
            pl.BlockSpec((D, D), lambda i: (0, 0)),
            pl.BlockSpec((D, D), lambda i: (0, 0)),
            pl.BlockSpec((1, D), lambda i: (0, 0)),
        ],
        out_specs=pl.BlockSpec((BN, D), lambda i: (i, 0)),
        out_shape=jax.ShapeDtypeStruct((N, D), jnp.float32),
    )(x_nodes, partials[:N], partials[N:], W_comb[:, :D].T, W_comb[:, D:].T,
      b_comb.reshape(1, D))

    return out


# speedup vs baseline: 5.3558x; 2.8823x over previous
import functools

import jax
import jax.numpy as jnp
import numpy as np
from jax import lax
from jax.experimental import pallas as pl
from jax.experimental.pallas import tpu as pltpu
from jax.experimental.pallas import tpu_sc as plsc

_CUTOFF = 5.0
_NC = 2
_NS = 16
_NW = _NC * _NS
_CH = 128


def _tgather_kernel(EW, nt_hbm, src_hbm, t_hbm, nt_v, src_v, t_v):
    c = lax.axis_index("c")
    s = lax.axis_index("s")
    w = s * _NC + c
    base = w * EW
    pltpu.sync_copy(nt_hbm, nt_v)
    pltpu.sync_copy(src_hbm.at[pl.ds(base, EW)], src_v)

    def body(i, carry):
        idx = src_v[pl.ds(i * 16, 16)]
        t_v[pl.ds(i * 16, 16)] = plsc.load_gather(nt_v, [idx])
        return carry

    lax.fori_loop(0, EW // 16, body, 0)
    pltpu.sync_copy(t_v, t_hbm.at[pl.ds(base, EW)])


def _msg_kernel(BE, rbf_ref, t_ref, dist_ref, WeT_ref, be_ref, embp_ref, out_ref):
    t = t_ref[...]
    d = dist_ref[...]
    w = 0.5 * (jnp.cos(d * (np.pi / _CUTOFF)) + 1.0)
    w = w * (d < _CUTOFF).astype(jnp.float32)
    onehot_t = (t == lax.broadcasted_iota(jnp.int32, (128, BE), 0)).astype(
        jnp.float32)
    onehot_t = onehot_t * w
    emb_rows_w = lax.dot_general(
        onehot_t, embp_ref[...], (((0,), (0,)), ((), ())),
        preferred_element_type=jnp.float32)
    xe = jnp.dot(rbf_ref[...], WeT_ref[...], preferred_element_type=jnp.float32)
    xe = xe + be_ref[...]
    out_ref[...] = emb_rows_w * xe


def _scatter_kernel(NCH, N, D, msg_hbm, dst_hbm, out_hbm, dst_v, msg_v, acc,
                    sem0, sem1):
    sem = [sem0, sem1]
    c = lax.axis_index("c")
    s = lax.axis_index("s")
    w = c * _NS + s
    zr = 624
    tail = N - zr * _NS

    def zb(i, carry):
        msg_v[0, i // (D // 16), pl.ds((i % (D // 16)) * 16, 16)] = jnp.zeros(
            (16,), jnp.float32)
        return carry

    lax.fori_loop(0, _CH * (D // 16), zb, 0)
    for kz in range(4):
        pltpu.sync_copy(msg_v.at[0], acc.at[pl.ds(s * zr + kz * _CH, _CH)])
    pltpu.sync_copy(msg_v.at[0, pl.ds(0, zr - 4 * _CH)],
                    acc.at[pl.ds(s * zr + 4 * _CH, zr - 4 * _CH)])

    @pl.when(s == 0)
    def _():
        pltpu.sync_copy(msg_v.at[0, pl.ds(0, tail)],
                        acc.at[pl.ds(_NS * zr, tail)])

    plsc.subcore_barrier()

    n_even = NCH // _NW
    n_extra = NCH - n_even * _NW
    chunk0 = w * n_even + jnp.minimum(w, n_extra)

    def mkmsg(k, b):
        return pltpu.make_async_copy(
            msg_hbm.at[pl.ds((chunk0 + k) * _CH, _CH)], msg_v.at[b], sem[b])

    def mkdst(k, b):
        return pltpu.make_async_copy(
            dst_hbm.at[pl.ds((chunk0 + k) * _CH, _CH)], dst_v.at[b], sem[b])

    for b in range(2):
        mkmsg(b, b).start()
        mkdst(b, b).start()

    def body(k2, carry):
        for b in range(2):
            k = k2 * 2 + b
            mkmsg(k, b).wait()
            mkdst(k, b).wait()
            pltpu.sync_copy(msg_v.at[b], acc.at[dst_v.at[b]], add=True)
            nk = jnp.minimum(k + 2, n_even - 1)
            mkmsg(nk, b).start()
            mkdst(nk, b).start()
        return carry

    lax.fori_loop(0, n_even // 2, body, 0)
    for b in range(2):
        mkmsg(0, b).wait()
        mkdst(0, b).wait()

    @pl.when(w < n_extra)
    def _():
        base = (chunk0 + n_even) * _CH
        pltpu.sync_copy(dst_hbm.at[pl.ds(base, _CH)], dst_v.at[0])
        pltpu.sync_copy(msg_hbm.at[pl.ds(base, _CH)], msg_v.at[0])
        pltpu.sync_copy(msg_v.at[0], acc.at[dst_v.at[0]], add=True)

    plsc.subcore_barrier()

    pltpu.sync_copy(acc.at[pl.ds(s * zr, zr)],
                    out_hbm.at[pl.ds(c * N + s * zr, zr)])

    @pl.when(s == 0)
    def _():
        pltpu.sync_copy(acc.at[pl.ds(_NS * zr, tail)],
                        out_hbm.at[pl.ds(c * N + _NS * zr, tail)])


def _comb_kernel(xn_ref, p0_ref, p1_ref, Wc1T_ref, Wc2T_ref, bc_ref, out_ref):
    ps = p0_ref[...] + p1_ref[...]
    out_ref[...] = (
        jnp.dot(xn_ref[...], Wc1T_ref[...], preferred_element_type=jnp.float32)
        + jnp.dot(ps, Wc2T_ref[...], preferred_element_type=jnp.float32)
        + bc_ref[...]
    )


def kernel(x_nodes, node_type, edge_index, rbf_edges, dist, emb_table,
           W_edge, b_edge, W_comb, b_comb):
    N, D = x_nodes.shape
    E = edge_index.shape[1]
    n_rbf = rbf_edges.shape[1]
    n_elem = emb_table.shape[0]

    src = edge_index[0].astype(jnp.int32)
    dst = edge_index[1].astype(jnp.int32)
    emb_pad = jnp.zeros((128, D), emb_table.dtype).at[:n_elem].set(emb_table)

    mesh = plsc.VectorSubcoreMesh(core_axis_name="c", subcore_axis_name="s")

    EW = E // _NW
    t = pl.kernel(
        functools.partial(_tgather_kernel, EW),
        out_type=jax.ShapeDtypeStruct((E,), jnp.int32),
        mesh=mesh,
        scratch_types=[
            pltpu.VMEM((N,), jnp.int32),
            pltpu.VMEM((EW,), jnp.int32),
            pltpu.VMEM((EW,), jnp.int32),
        ],
        compiler_params=pltpu.CompilerParams(needs_layout_passes=False),
    )(node_type.astype(jnp.int32), src)

    BE = 3200
    msg = pl.pallas_call(
        functools.partial(_msg_kernel, BE),
        grid=(E // BE,),
        in_specs=[
            pl.BlockSpec((BE, n_rbf), lambda i: (i, 0)),
            pl.BlockSpec((1, BE), lambda i: (0, i)),
            pl.BlockSpec((1, BE), lambda i: (0, i)),
            pl.BlockSpec((n_rbf, D), lambda i: (0, 0)),
            pl.BlockSpec((1, D), lambda i: (0, 0)),
            pl.BlockSpec((128, D), lambda i: (0, 0)),
        ],
        out_specs=pl.BlockSpec((BE, D), lambda i: (i, 0)),
        out_shape=jax.ShapeDtypeStruct((E, D), jnp.float32),
    )(rbf_edges, t.reshape(1, E), dist.reshape(1, E),
      W_edge.T, b_edge.reshape(1, D), emb_pad)

    NCH = E // _CH
    partials = pl.kernel(
        functools.partial(_scatter_kernel, NCH, N, D),
        out_type=jax.ShapeDtypeStruct((_NC * N, D), jnp.float32),
        mesh=mesh,
        scratch_types=[
            pltpu.VMEM((2, _CH), jnp.int32),
            pltpu.VMEM((2, _CH, D), jnp.float32),
            pltpu.VMEM_SHARED((N, D), jnp.float32),
            pltpu.SemaphoreType.DMA,
            pltpu.SemaphoreType.DMA,
        ],
    )(msg, dst)

    BN = 2000
    nb = N // BN
    out = pl.pallas_call(
        _comb_kernel,
        grid=(nb,),
        in_specs=[
            pl.BlockSpec((BN, D), lambda i: (i, 0)),
            pl.BlockSpec((BN, D), lambda i: (i, 0)),
            pl.BlockSpec((BN, D), lambda i, nb=nb: (i + nb, 0)),
            pl.BlockSpec((D, D), lambda i: (0, 0)),
            pl.BlockSpec((D, D), lambda i: (0, 0)),
            pl.BlockSpec((1, D), lambda i: (0, 0)),
        ],
        out_specs=pl.BlockSpec((BN, D), lambda i: (i, 0)),
        out_shape=jax.ShapeDtypeStruct((N, D), jnp.float32),
    )(x_nodes, partials, partials, W_comb[:, :D].T, W_comb[:, D:].T,
      b_comb.reshape(1, D))

    return out


# speedup vs baseline: 7.4934x; 1.3991x over previous
import functools

import jax
import jax.numpy as jnp
import numpy as np
from jax import lax
from jax.experimental import pallas as pl
from jax.experimental.pallas import tpu as pltpu
from jax.experimental.pallas import tpu_sc as plsc

_CUTOFF = 5.0
_NC = 2
_NS = 16
_NW = _NC * _NS
_CH = 128


def _tgather_kernel(NCH, nt_hbm, src_hbm, d_hbm, t_hbm, d2_hbm, nt_v, src_v,
                    t_v, d_v):
    c = lax.axis_index("c")
    s = lax.axis_index("s")
    w = s * _NC + c
    n_even = NCH // _NW
    n_extra = NCH - n_even * _NW
    chunk0 = w * n_even + jnp.minimum(w, n_extra)
    pltpu.sync_copy(nt_hbm, nt_v)

    def gathered(base, cnt):
        pltpu.sync_copy(src_hbm.at[pl.ds(base, cnt)], src_v.at[pl.ds(0, cnt)])

        def body(i, carry):
            idx = src_v[pl.ds(i * 16, 16)]
            t_v[pl.ds(i * 16, 16)] = plsc.load_gather(nt_v, [idx])
            return carry

        lax.fori_loop(0, cnt // 16, body, 0)
        pltpu.sync_copy(t_v.at[pl.ds(0, cnt)], t_hbm.at[0, pl.ds(base, cnt)])
        pltpu.sync_copy(d_hbm.at[pl.ds(base, cnt)], d_v.at[pl.ds(0, cnt)])
        pltpu.sync_copy(d_v.at[pl.ds(0, cnt)], d2_hbm.at[0, pl.ds(base, cnt)])

    gathered(chunk0 * _CH, n_even * _CH)

    @pl.when(w < n_extra)
    def _():
        gathered((chunk0 + n_even) * _CH, _CH)


def _msg_kernel(BE, rbfT_ref, t_ref, dist_ref, WeT_ref, be_ref, embp_ref, out_ref):
    t = t_ref[...]
    d = dist_ref[...]
    w = 0.5 * (jnp.cos(d * (np.pi / _CUTOFF)) + 1.0)
    w = w * (d < _CUTOFF).astype(jnp.float32)
    onehot_t = (t == lax.broadcasted_iota(jnp.int32, (128, BE), 0)).astype(
        jnp.float32)
    onehot_t = onehot_t * w
    emb_rows_w = lax.dot_general(
        onehot_t, embp_ref[...], (((0,), (0,)), ((), ())),
        preferred_element_type=jnp.float32)
    xe = lax.dot_general(
        rbfT_ref[...], WeT_ref[...], (((0,), (0,)), ((), ())),
        preferred_element_type=jnp.float32)
    xe = xe + be_ref[...]
    out_ref[...] = emb_rows_w * xe


def _scatter_kernel(NCH, N, D, msg_hbm, dst_hbm, out_hbm, dst_v, msg_v, acc,
                    sem0, sem1):
    sem = [sem0, sem1]
    c = lax.axis_index("c")
    s = lax.axis_index("s")
    w = c * _NS + s
    zr = 624
    tail = N - zr * _NS

    def zb(i, carry):
        msg_v[0, i // (D // 16), pl.ds((i % (D // 16)) * 16, 16)] = jnp.zeros(
            (16,), jnp.float32)
        return carry

    lax.fori_loop(0, _CH * (D // 16), zb, 0)
    for kz in range(4):
        pltpu.sync_copy(msg_v.at[0], acc.at[pl.ds(s * zr + kz * _CH, _CH)])
    pltpu.sync_copy(msg_v.at[0, pl.ds(0, zr - 4 * _CH)],
                    acc.at[pl.ds(s * zr + 4 * _CH, zr - 4 * _CH)])

    @pl.when(s == 0)
    def _():
        pltpu.sync_copy(msg_v.at[0, pl.ds(0, tail)],
                        acc.at[pl.ds(_NS * zr, tail)])

    plsc.subcore_barrier()

    n_even = NCH // _NW
    n_extra = NCH - n_even * _NW
    chunk0 = w * n_even + jnp.minimum(w, n_extra)

    def mkmsg(k, b):
        return pltpu.make_async_copy(
            msg_hbm.at[pl.ds((chunk0 + k) * _CH, _CH)], msg_v.at[b], sem[b])

    def mkdst(k, b):
        return pltpu.make_async_copy(
            dst_hbm.at[pl.ds((chunk0 + k) * _CH, _CH)], dst_v.at[b], sem[b])

    for b in range(2):
        mkmsg(b, b).start()
        mkdst(b, b).start()

    def body(k2, carry):
        for b in range(2):
            k = k2 * 2 + b
            mkmsg(k, b).wait()
            mkdst(k, b).wait()
            pltpu.sync_copy(msg_v.at[b], acc.at[dst_v.at[b]], add=True)
            nk = jnp.minimum(k + 2, n_even - 1)
            mkmsg(nk, b).start()
            mkdst(nk, b).start()
        return carry

    lax.fori_loop(0, n_even // 2, body, 0)
    for b in range(2):
        mkmsg(0, b).wait()
        mkdst(0, b).wait()

    @pl.when(w < n_extra)
    def _():
        base = (chunk0 + n_even) * _CH
        pltpu.sync_copy(dst_hbm.at[pl.ds(base, _CH)], dst_v.at[0])
        pltpu.sync_copy(msg_hbm.at[pl.ds(base, _CH)], msg_v.at[0])
        pltpu.sync_copy(msg_v.at[0], acc.at[dst_v.at[0]], add=True)

    plsc.subcore_barrier()

    pltpu.sync_copy(acc.at[pl.ds(s * zr, zr)],
                    out_hbm.at[pl.ds(c * N + s * zr, zr)])

    @pl.when(s == 0)
    def _():
        pltpu.sync_copy(acc.at[pl.ds(_NS * zr, tail)],
                        out_hbm.at[pl.ds(c * N + _NS * zr, tail)])


def _comb_kernel(xn_ref, p0_ref, p1_ref, Wc1T_ref, Wc2T_ref, bc_ref, out_ref):
    ps = p0_ref[...] + p1_ref[...]
    out_ref[...] = (
        jnp.dot(xn_ref[...], Wc1T_ref[...], preferred_element_type=jnp.float32)
        + jnp.dot(ps, Wc2T_ref[...], preferred_element_type=jnp.float32)
        + bc_ref[...]
    )


def kernel(x_nodes, node_type, edge_index, rbf_edges, dist, emb_table,
           W_edge, b_edge, W_comb, b_comb):
    N, D = x_nodes.shape
    E = edge_index.shape[1]
    n_rbf = rbf_edges.shape[1]
    n_elem = emb_table.shape[0]

    src = edge_index[0].astype(jnp.int32)
    dst = edge_index[1].astype(jnp.int32)
    emb_pad = jnp.zeros((128, D), emb_table.dtype).at[:n_elem].set(emb_table)

    mesh = plsc.VectorSubcoreMesh(core_axis_name="c", subcore_axis_name="s")

    NCH = E // _CH
    EW = (NCH // _NW) * _CH
    t, dist2 = pl.kernel(
        functools.partial(_tgather_kernel, NCH),
        out_type=[jax.ShapeDtypeStruct((1, E), jnp.int32),
                  jax.ShapeDtypeStruct((1, E), jnp.float32)],
        mesh=mesh,
        scratch_types=[
            pltpu.VMEM((N,), jnp.int32),
            pltpu.VMEM((EW,), jnp.int32),
            pltpu.VMEM((EW,), jnp.int32),
            pltpu.VMEM((EW,), jnp.float32),
        ],
        compiler_params=pltpu.CompilerParams(needs_layout_passes=False),
    )(node_type.astype(jnp.int32), src, dist)

    BE = 3200
    msg = pl.pallas_call(
        functools.partial(_msg_kernel, BE),
        grid=(E // BE,),
        in_specs=[
            pl.BlockSpec((n_rbf, BE), lambda i: (0, i)),
            pl.BlockSpec((1, BE), lambda i: (0, i)),
            pl.BlockSpec((1, BE), lambda i: (0, i)),
            pl.BlockSpec((n_rbf, D), lambda i: (0, 0)),
            pl.BlockSpec((1, D), lambda i: (0, 0)),
            pl.BlockSpec((128, D), lambda i: (0, 0)),
        ],
        out_specs=pl.BlockSpec((BE, D), lambda i: (i, 0)),
        out_shape=jax.ShapeDtypeStruct((E, D), jnp.float32),
    )(rbf_edges.T, t, dist2, W_edge.T, b_edge.reshape(1, D), emb_pad)

    partials = pl.kernel(
        functools.partial(_scatter_kernel, NCH, N, D),
        out_type=jax.ShapeDtypeStruct((_NC * N, D), jnp.float32),
        mesh=mesh,
        scratch_types=[
            pltpu.VMEM((2, _CH), jnp.int32),
            pltpu.VMEM((2, _CH, D), jnp.float32),
            pltpu.VMEM_SHARED((N, D), jnp.float32),
            pltpu.SemaphoreType.DMA,
            pltpu.SemaphoreType.DMA,
        ],
    )(msg, dst)

    BN = 2000
    nb = N // BN
    out = pl.pallas_call(
        _comb_kernel,
        grid=(nb,),
        in_specs=[
            pl.BlockSpec((BN, D), lambda i: (i, 0)),
            pl.BlockSpec((BN, D), lambda i: (i, 0)),
            pl.BlockSpec((BN, D), lambda i, nb=nb: (i + nb, 0)),
            pl.BlockSpec((D, D), lambda i: (0, 0)),
            pl.BlockSpec((D, D), lambda i: (0, 0)),
            pl.BlockSpec((1, D), lambda i: (0, 0)),
        ],
        out_specs=pl.BlockSpec((BN, D), lambda i: (i, 0)),
        out_shape=jax.ShapeDtypeStruct((N, D), jnp.float32),
    )(x_nodes, partials, partials, W_comb[:, :D].T, W_comb[:, D:].T,
      b_comb.reshape(1, D))

    return out


# speedup vs baseline: 7.7767x; 1.0378x over previous
import functools

import jax
import jax.numpy as jnp
import numpy as np
from jax import lax
from jax.experimental import pallas as pl
from jax.experimental.pallas import tpu as pltpu
from jax.experimental.pallas import tpu_sc as plsc

_CUTOFF = 5.0
_NC = 2
_NS = 16
_NW = _NC * _NS
_CH = 128


def _tgather_kernel(NCH, nt_hbm, ei_hbm, d_hbm, t_hbm, d2_hbm, nt_v, ei_v,
                    t_v, d_v):
    c = lax.axis_index("c")
    s = lax.axis_index("s")
    w = s * _NC + c
    n_even = NCH // _NW
    n_extra = NCH - n_even * _NW
    chunk0 = w * n_even + jnp.minimum(w, n_extra)
    pltpu.sync_copy(nt_hbm, nt_v)

    def gathered(base, cnt):
        pltpu.sync_copy(ei_hbm.at[:, pl.ds(base, cnt)],
                        ei_v.at[:, pl.ds(0, cnt)])

        def body(i, carry):
            idx = ei_v[0, pl.ds(i * 16, 16)]
            t_v[pl.ds(i * 16, 16)] = plsc.load_gather(nt_v, [idx])
            return carry

        lax.fori_loop(0, cnt // 16, body, 0)
        pltpu.sync_copy(t_v.at[pl.ds(0, cnt)], t_hbm.at[0, pl.ds(base, cnt)])
        pltpu.sync_copy(d_hbm.at[pl.ds(base, cnt)], d_v.at[pl.ds(0, cnt)])
        pltpu.sync_copy(d_v.at[pl.ds(0, cnt)], d2_hbm.at[0, pl.ds(base, cnt)])

    gathered(chunk0 * _CH, n_even * _CH)

    @pl.when(w < n_extra)
    def _():
        gathered((chunk0 + n_even) * _CH, _CH)


def _msg_kernel(BE, rbfT_ref, t_ref, dist_ref, WeT_ref, be_ref, embp_ref, out_ref):
    t = t_ref[...]
    d = dist_ref[...]
    w = 0.5 * (jnp.cos(d * (np.pi / _CUTOFF)) + 1.0)
    w = w * (d < _CUTOFF).astype(jnp.float32)
    onehot_t = (t == lax.broadcasted_iota(jnp.int32, (128, BE), 0)).astype(
        jnp.float32)
    onehot_t = onehot_t * w
    emb_rows_w = lax.dot_general(
        onehot_t, embp_ref[...], (((0,), (0,)), ((), ())),
        preferred_element_type=jnp.float32)
    xe = lax.dot_general(
        rbfT_ref[...], WeT_ref[...], (((0,), (0,)), ((), ())),
        preferred_element_type=jnp.float32)
    xe = xe + be_ref[...]
    out_ref[...] = emb_rows_w * xe


def _scatter_kernel(NCH, N, D, msg_hbm, dst_hbm, out_hbm, dst_v, msg_v, acc,
                    sem0, sem1):
    sem = [sem0, sem1]
    c = lax.axis_index("c")
    s = lax.axis_index("s")
    w = c * _NS + s
    zr = 624
    tail = N - zr * _NS

    def zb(i, carry):
        msg_v[0, i // (D // 16), pl.ds((i % (D // 16)) * 16, 16)] = jnp.zeros(
            (16,), jnp.float32)
        return carry

    lax.fori_loop(0, _CH * (D // 16), zb, 0)
    for kz in range(4):
        pltpu.sync_copy(msg_v.at[0], acc.at[pl.ds(s * zr + kz * _CH, _CH)])
    pltpu.sync_copy(msg_v.at[0, pl.ds(0, zr - 4 * _CH)],
                    acc.at[pl.ds(s * zr + 4 * _CH, zr - 4 * _CH)])

    @pl.when(s == 0)
    def _():
        pltpu.sync_copy(msg_v.at[0, pl.ds(0, tail)],
                        acc.at[pl.ds(_NS * zr, tail)])

    plsc.subcore_barrier()

    n_even = NCH // _NW
    n_extra = NCH - n_even * _NW
    chunk0 = w * n_even + jnp.minimum(w, n_extra)

    def mkmsg(k, b):
        return pltpu.make_async_copy(
            msg_hbm.at[pl.ds((chunk0 + k) * _CH, _CH)], msg_v.at[b], sem[b])

    def mkdst(k, b):
        return pltpu.make_async_copy(
            dst_hbm.at[:, pl.ds((chunk0 + k) * _CH, _CH)], dst_v.at[b], sem[b])

    for b in range(2):
        mkmsg(b, b).start()
        mkdst(b, b).start()

    def body(k2, carry):
        for b in range(2):
            k = k2 * 2 + b
            mkmsg(k, b).wait()
            mkdst(k, b).wait()
            pltpu.sync_copy(msg_v.at[b], acc.at[dst_v.at[b, 1]], add=True)
            nk = jnp.minimum(k + 2, n_even - 1)
            mkmsg(nk, b).start()
            mkdst(nk, b).start()
        return carry

    lax.fori_loop(0, n_even // 2, body, 0)
    for b in range(2):
        mkmsg(0, b).wait()
        mkdst(0, b).wait()

    @pl.when(w < n_extra)
    def _():
        base = (chunk0 + n_even) * _CH
        pltpu.sync_copy(dst_hbm.at[:, pl.ds(base, _CH)], dst_v.at[0])
        pltpu.sync_copy(msg_hbm.at[pl.ds(base, _CH)], msg_v.at[0])
        pltpu.sync_copy(msg_v.at[0], acc.at[dst_v.at[0, 1]], add=True)

    plsc.subcore_barrier()

    pltpu.sync_copy(acc.at[pl.ds(s * zr, zr)],
                    out_hbm.at[pl.ds(c * N + s * zr, zr)])

    @pl.when(s == 0)
    def _():
        pltpu.sync_copy(acc.at[pl.ds(_NS * zr, tail)],
                        out_hbm.at[pl.ds(c * N + _NS * zr, tail)])


def _comb_kernel(xn_ref, p0_ref, p1_ref, Wc1T_ref, Wc2T_ref, bc_ref, out_ref):
    ps = p0_ref[...] + p1_ref[...]
    out_ref[...] = (
        jnp.dot(xn_ref[...], Wc1T_ref[...], preferred_element_type=jnp.float32)
        + jnp.dot(ps, Wc2T_ref[...], preferred_element_type=jnp.float32)
        + bc_ref[...]
    )


def kernel(x_nodes, node_type, edge_index, rbf_edges, dist, emb_table,
           W_edge, b_edge, W_comb, b_comb):
    N, D = x_nodes.shape
    E = edge_index.shape[1]
    n_rbf = rbf_edges.shape[1]
    n_elem = emb_table.shape[0]

    ei = edge_index.astype(jnp.int32)
    emb_pad = jnp.zeros((128, D), emb_table.dtype).at[:n_elem].set(emb_table)

    mesh = plsc.VectorSubcoreMesh(core_axis_name="c", subcore_axis_name="s")

    NCH = E // _CH
    EW = (NCH // _NW) * _CH
    t, dist2 = pl.kernel(
        functools.partial(_tgather_kernel, NCH),
        out_type=[jax.ShapeDtypeStruct((1, E), jnp.int32),
                  jax.ShapeDtypeStruct((1, E), jnp.float32)],
        mesh=mesh,
        scratch_types=[
            pltpu.VMEM((N,), jnp.int32),
            pltpu.VMEM((2, EW), jnp.int32),
            pltpu.VMEM((EW,), jnp.int32),
            pltpu.VMEM((EW,), jnp.float32),
        ],
        compiler_params=pltpu.CompilerParams(needs_layout_passes=False),
    )(node_type.astype(jnp.int32), ei, dist)

    BE = 3200
    msg = pl.pallas_call(
        functools.partial(_msg_kernel, BE),
        grid=(E // BE,),
        in_specs=[
            pl.BlockSpec((n_rbf, BE), lambda i: (0, i)),
            pl.BlockSpec((1, BE), lambda i: (0, i)),
            pl.BlockSpec((1, BE), lambda i: (0, i)),
            pl.BlockSpec((n_rbf, D), lambda i: (0, 0)),
            pl.BlockSpec((1, D), lambda i: (0, 0)),
            pl.BlockSpec((128, D), lambda i: (0, 0)),
        ],
        out_specs=pl.BlockSpec((BE, D), lambda i: (i, 0)),
        out_shape=jax.ShapeDtypeStruct((E, D), jnp.float32),
    )(rbf_edges.T, t, dist2, W_edge.T, b_edge.reshape(1, D), emb_pad)

    partials = pl.kernel(
        functools.partial(_scatter_kernel, NCH, N, D),
        out_type=jax.ShapeDtypeStruct((_NC * N, D), jnp.float32),
        mesh=mesh,
        scratch_types=[
            pltpu.VMEM((2, 2, _CH), jnp.int32),
            pltpu.VMEM((2, _CH, D), jnp.float32),
            pltpu.VMEM_SHARED((N, D), jnp.float32),
            pltpu.SemaphoreType.DMA,
            pltpu.SemaphoreType.DMA,
        ],
    )(msg, ei)

    BN = 2000
    nb = N // BN
    out = pl.pallas_call(
        _comb_kernel,
        grid=(nb,),
        in_specs=[
            pl.BlockSpec((BN, D), lambda i: (i, 0)),
            pl.BlockSpec((BN, D), lambda i: (i, 0)),
            pl.BlockSpec((BN, D), lambda i, nb=nb: (i + nb, 0)),
            pl.BlockSpec((D, D), lambda i: (0, 0)),
            pl.BlockSpec((D, D), lambda i: (0, 0)),
            pl.BlockSpec((1, D), lambda i: (0, 0)),
        ],
        out_specs=pl.BlockSpec((BN, D), lambda i: (i, 0)),
        out_shape=jax.ShapeDtypeStruct((N, D), jnp.float32),
    )(x_nodes, partials, partials, W_comb[:, :D].T, W_comb[:, D:].T,
      b_comb.reshape(1, D))

    return out
